# baseline (speedup 1.0000x reference)
import functools

import jax
import jax.numpy as jnp
from jax.experimental import pallas as pl
from jax.experimental.pallas import tpu as pltpu

B, S, D_IN, M, NS = 8, 32, 1024, 256, 64
UPDATE_RATE = 0.5
BH = B // 2


def _mc_kernel(hs_ref, mem0_ref, winT_ref, wvalT_ref, wrg1_ref, wrg2_ref,
               wu1_ref, wu2_ref, b_in_ref, b_val_ref, b_rg_ref, b_u_ref,
               out_ref, memin_ref, xrg_ref, xu_ref):
    f32 = jnp.float32
    bf16 = jnp.bfloat16
    hs = hs_ref[:]
    memin_ref[:] = jnp.dot(hs, winT_ref[:], preferred_element_type=f32) + b_in_ref[:]
    val = (jnp.dot(hs, wvalT_ref[:], preferred_element_type=f32)
           + b_val_ref[:]).astype(bf16)
    xrg_ref[:] = jnp.dot(val, wrg1_ref[:], preferred_element_type=f32) + b_rg_ref[:]
    xu_ref[:] = jnp.dot(val, wu1_ref[:], preferred_element_type=f32) + b_u_ref[:]
    out_ref[:] = mem0_ref[:]
    wrg2 = wrg2_ref[:]
    wu2 = wu2_ref[:]
    def step(t, usage):
        memin_t = memin_ref[pl.ds(t * B, B), :]
        xrg_t = xrg_ref[pl.ds(t * B, B), :]
        xu_t = xu_ref[pl.ds(t * B, B), :]
        mem = out_ref[:]
        mem_bf = mem.astype(bf16)
        mem3 = mem.reshape(B, NS, M)
        sim = jnp.sum(mem3 * memin_t[:, None, :], axis=-1)
        w = jax.nn.softmax(0.2 * usage - sim, axis=-1)
        w_eff = jnp.where(w > 0.01, w, 0.0)

        rg = jnp.dot(mem_bf, wrg2, preferred_element_type=f32)
        rg = jax.nn.sigmoid(rg.reshape(B, NS, 2 * M) + xrg_t[:, None, :])
        r = rg[:, :, :M]
        g_ = rg[:, :, M:]
        rm = (r.astype(bf16) * mem_bf.reshape(B, NS, M)).reshape(B * NS, M)
        c = jnp.dot(rm, wu2, preferred_element_type=f32).reshape(B, NS, M)
        c = jnp.tanh(c + xu_t[:, None, :])

        ag = (w_eff * UPDATE_RATE)[:, :, None] * g_
        mem_new = mem3 + ag * (c - mem3)
        nsq = jnp.sum(mem_new * mem_new, axis=-1, keepdims=True)
        mem_new = mem_new * jax.lax.rsqrt(jnp.maximum(nsq, 1e-24))
        out_ref[:] = mem_new.reshape(B * NS, M)
        return (usage + w_eff) * 0.99

    jax.lax.fori_loop(0, S, step, jnp.zeros((B, NS), f32), unroll=True)


@functools.partial(jax.jit, static_argnames=())
def kernel(hidden_states, memory_init, W_in, b_in, W_val, b_val,
           W_reset, b_reset, W_gate, b_gate, W_update, b_update):
    f32 = jnp.float32
    bf16 = jnp.bfloat16
    hs2 = hidden_states.transpose(1, 0, 2).reshape(S * B, D_IN).astype(bf16)
    mem0 = memory_init.reshape(B * NS, M)
    winT = W_in.T.astype(bf16)
    wvalT = W_val.T.astype(bf16)
    wrg1 = jnp.concatenate([W_reset[:, :M].T, W_gate[:, :M].T], axis=1).astype(bf16)
    wrg2 = jnp.concatenate([W_reset[:, M:].T, W_gate[:, M:].T], axis=1).astype(bf16)
    wu1 = W_update[:, :M].T.astype(bf16)
    wu2 = W_update[:, M:].T.astype(bf16)
    b_rg = jnp.concatenate([b_reset, b_gate]).reshape(1, 2 * M)

    out = pl.pallas_call(
        _mc_kernel,
        out_shape=jax.ShapeDtypeStruct((B * NS, M), f32),
        scratch_shapes=[
            pltpu.VMEM((S * B, M), f32),
            pltpu.VMEM((S * B, 2 * M), f32),
            pltpu.VMEM((S * B, M), f32),
        ],
    )(hs2, mem0, winT, wvalT, wrg1, wrg2, wu1, wu2,
      b_in.reshape(1, M), b_val.reshape(1, M), b_rg, b_update.reshape(1, M))
    return out.reshape(B, NS, M)


# speedup vs baseline: 1.0858x; 1.0858x over previous
import functools

import jax
import jax.numpy as jnp
from jax.experimental import pallas as pl
from jax.experimental.pallas import tpu as pltpu

B, S, D_IN, M, NS = 8, 32, 1024, 256, 64
UPDATE_RATE = 0.5


def _mc_kernel(hs_ref, mem0_ref, winT_ref, wvalT_ref, wrg1_ref, wrg2_ref,
               wu1_ref, wu2_ref, b_in_ref, b_val_ref, b_rg_ref, b_u_ref,
               out_ref, memin_ref, xrg_ref, xu_ref):
    f32 = jnp.float32
    bf16 = jnp.bfloat16
    hs = hs_ref[:]
    memin_ref[:] = jnp.dot(hs, winT_ref[:], preferred_element_type=f32) + b_in_ref[:]
    val = (jnp.dot(hs, wvalT_ref[:], preferred_element_type=f32)
           + b_val_ref[:]).astype(bf16)
    xrg_ref[:] = jnp.dot(val, wrg1_ref[:], preferred_element_type=f32) + b_rg_ref[:]
    xu_ref[:] = jnp.dot(val, wu1_ref[:], preferred_element_type=f32) + b_u_ref[:]
    out_ref[:] = mem0_ref[:]
    wr2 = wrg2_ref[:, :M]
    wg2 = wrg2_ref[:, M:]
    wu2 = wu2_ref[:]
    def step(t, usage):
        memin_t = memin_ref[pl.ds(t * B, B), :]
        xrg_t = xrg_ref[pl.ds(t * B, B), :]
        xu_t = xu_ref[pl.ds(t * B, B), :]
        mem = out_ref[:]
        mem_bf = mem.astype(bf16)
        mem3 = mem.reshape(B, NS, M)
        sim = jnp.sum(mem3 * memin_t[:, None, :], axis=-1)
        w = jax.nn.softmax(0.2 * usage - sim, axis=-1)
        w_eff = jnp.where(w > 0.01, w, 0.0)

        r = jnp.dot(mem_bf, wr2, preferred_element_type=f32)
        r = jax.nn.sigmoid(r.reshape(B, NS, M) + xrg_t[:, None, :M])
        rm = (r.astype(bf16) * mem_bf.reshape(B, NS, M)).reshape(B * NS, M)
        c = jnp.dot(rm, wu2, preferred_element_type=f32).reshape(B, NS, M)
        c = jnp.tanh(c + xu_t[:, None, :])
        g_ = jnp.dot(mem_bf, wg2, preferred_element_type=f32)
        g_ = jax.nn.sigmoid(g_.reshape(B, NS, M) + xrg_t[:, None, M:])

        ag = (w_eff * UPDATE_RATE)[:, :, None] * g_
        mem_new = mem3 + ag * (c - mem3)
        nsq = jnp.sum(mem_new * mem_new, axis=-1, keepdims=True)
        mem_new = mem_new * jax.lax.rsqrt(jnp.maximum(nsq, 1e-24))
        out_ref[:] = mem_new.reshape(B * NS, M)
        return (usage + w_eff) * 0.99

    jax.lax.fori_loop(0, S, step, jnp.zeros((B, NS), f32), unroll=True)


@functools.partial(jax.jit, static_argnames=())
def kernel(hidden_states, memory_init, W_in, b_in, W_val, b_val,
           W_reset, b_reset, W_gate, b_gate, W_update, b_update):
    f32 = jnp.float32
    bf16 = jnp.bfloat16
    hs2 = hidden_states.transpose(1, 0, 2).reshape(S * B, D_IN).astype(bf16)
    mem0 = memory_init.reshape(B * NS, M)
    winT = W_in.T.astype(bf16)
    wvalT = W_val.T.astype(bf16)
    wrg1 = jnp.concatenate([W_reset[:, :M].T, W_gate[:, :M].T], axis=1).astype(bf16)
    wrg2 = jnp.concatenate([W_reset[:, M:].T, W_gate[:, M:].T], axis=1).astype(bf16)
    wu1 = W_update[:, :M].T.astype(bf16)
    wu2 = W_update[:, M:].T.astype(bf16)
    b_rg = jnp.concatenate([b_reset, b_gate]).reshape(1, 2 * M)

    out = pl.pallas_call(
        _mc_kernel,
        out_shape=jax.ShapeDtypeStruct((B * NS, M), f32),
        scratch_shapes=[
            pltpu.VMEM((S * B, M), f32),
            pltpu.VMEM((S * B, 2 * M), f32),
            pltpu.VMEM((S * B, M), f32),
        ],
    )(hs2, mem0, winT, wvalT, wrg1, wrg2, wu1, wu2,
      b_in.reshape(1, M), b_val.reshape(1, M), b_rg, b_update.reshape(1, M))
    return out.reshape(B, NS, M)
